# baseline (speedup 1.0000x reference)
import functools

import jax
import jax.numpy as jnp
from jax import lax
from jax.experimental import pallas as pl
from jax.experimental.pallas import tpu as pltpu
from jax.experimental.pallas import tpu_sc as plsc

IMP_K = 512
RECENT = 512
CACHE = IMP_K + RECENT
B, H, Q, S, D = 8, 16, 4, 4096, 128
SEL = S - RECENT
ROWS, LANES = 32, 128
SROWS = SEL // LANES
KHI = IMP_K // LANES


def _tc_select_body(attn_ref, idx_ref, imp_ref, cnt_ref):
    b = pl.program_id(0)
    a = attn_ref[...]
    imp2d = (jnp.sum(jnp.sum(a, axis=2), axis=1) / H)[0]

    row_io = lax.broadcasted_iota(jnp.int32, (ROWS, LANES), 0)
    col_io = lax.broadcasted_iota(jnp.int32, (ROWS, LANES), 1)
    sidx = row_io * LANES + col_io
    sel = sidx < SEL

    key = lax.bitcast_convert_type(imp2d, jnp.int32)
    keym = jnp.where(sel, key, jnp.int32(-1))

    def bit_step(i, t):
        cand = t | (jnp.int32(1) << (jnp.int32(30) - i))
        cnt = jnp.sum((keym >= cand).astype(jnp.int32))
        return jnp.where(cnt >= IMP_K, cand, t)

    thr = lax.fori_loop(0, 31, bit_step, jnp.int32(0))

    gt = keym > thr
    eq = keym == thr
    n_eq_take = (jnp.int32(IMP_K) - jnp.sum(gt.astype(jnp.int32))).astype(
        jnp.float32)

    strict_u = (lax.broadcasted_iota(jnp.int32, (LANES, LANES), 0)
                < lax.broadcasted_iota(jnp.int32, (LANES, LANES), 1)
                ).astype(jnp.float32)
    row_lt = (lax.broadcasted_iota(jnp.int32, (ROWS, ROWS), 1)
              < lax.broadcasted_iota(jnp.int32, (ROWS, ROWS), 0)
              ).astype(jnp.float32)

    def eprefix(m):
        within = jnp.dot(m, strict_u, preferred_element_type=jnp.float32,
                         precision=lax.Precision.HIGHEST)
        offs = jnp.dot(row_lt, jnp.sum(m, axis=1, keepdims=True),
                       preferred_element_type=jnp.float32,
                       precision=lax.Precision.HIGHEST)
        return within + offs

    tie_rank = eprefix(eq.astype(jnp.float32))
    keep = gt | (eq & (tie_rank < n_eq_take))
    rank = eprefix(keep.astype(jnp.float32)).astype(jnp.int32)

    rank_hi = rank >> 7
    rank_lo = jnp.where(keep, rank & 127, jnp.int32(-1))
    rank_hi_f = rank_hi.astype(jnp.float32)
    rank_lo_f = rank_lo.astype(jnp.float32)

    key_b0 = (key & 255).astype(jnp.float32)
    key_b1 = ((key >> 8) & 255).astype(jnp.float32)
    key_b2 = ((key >> 16) & 255).astype(jnp.float32)
    key_b3 = ((key >> 24) & 127).astype(jnp.float32)
    col_f = col_io.astype(jnp.float32)

    jlo_io = (lax.broadcasted_iota(jnp.int32, (LANES, LANES), 0)
              ).astype(jnp.float32)

    accs = [jnp.zeros((6, LANES), jnp.float32) for _ in range(KHI)]
    for i in range(SROWS):
        rlo = rank_lo_f[i:i + 1, :]
        rhi = rank_hi_f[i:i + 1, :]
        oneh = jnp.where(jlo_io == rlo, 1.0, 0.0)
        pay = jnp.concatenate(
            [jnp.full((1, LANES), float(i), jnp.float32),
             col_f[i:i + 1, :],
             key_b0[i:i + 1, :], key_b1[i:i + 1, :],
             key_b2[i:i + 1, :], key_b3[i:i + 1, :]], axis=0)
        for g in range(KHI):
            pay_g = pay * jnp.where(rhi == float(g), 1.0, 0.0)
            accs[g] = accs[g] + lax.dot_general(
                pay_g, oneh, (((1,), (1,)), ((), ())),
                preferred_element_type=jnp.float32)

    ch = [jnp.concatenate([accs[g][c:c + 1, :] for g in range(KHI)], axis=0)
          for c in range(6)]
    kidx_top = (ch[0].astype(jnp.int32) * LANES + ch[1].astype(jnp.int32))
    keybits = (ch[2].astype(jnp.int32)
               | (ch[3].astype(jnp.int32) << 8)
               | (ch[4].astype(jnp.int32) << 16)
               | (ch[5].astype(jnp.int32) << 24))
    imp_top = lax.bitcast_convert_type(keybits, jnp.float32)

    kidx8 = jnp.concatenate([kidx_top, sidx[SROWS:, :]], axis=0)
    imp8 = jnp.concatenate([imp_top, imp2d[SROWS:, :]], axis=0)

    h_io = lax.broadcasted_iota(jnp.int32, (H, 1, 1), 0)
    flat = (b * H + h_io) * S + kidx_top[None]
    idx_ref[...] = flat[None]
    imp_ref[...] = imp8[None]
    cnt_ref[...] = (jnp.float32(S) - kidx8.astype(jnp.float32))[None]


def _tc_select(attn_r):
    return pl.pallas_call(
        _tc_select_body,
        grid=(B,),
        in_specs=[pl.BlockSpec((1, H, Q, ROWS, LANES),
                               lambda b: (b, 0, 0, 0, 0))],
        out_specs=[
            pl.BlockSpec((1, H, KHI, LANES), lambda b: (b, 0, 0, 0)),
            pl.BlockSpec((1, CACHE // LANES, LANES), lambda b: (b, 0, 0)),
            pl.BlockSpec((1, CACHE // LANES, LANES), lambda b: (b, 0, 0)),
        ],
        out_shape=[
            jax.ShapeDtypeStruct((B, H, KHI, LANES), jnp.int32),
            jax.ShapeDtypeStruct((B, CACHE // LANES, LANES), jnp.float32),
            jax.ShapeDtypeStruct((B, CACHE // LANES, LANES), jnp.float32),
        ],
    )(attn_r)


_NC, _NS = 2, 16
_NW = _NC * _NS
_NROWS = B * H * IMP_K // LANES
_RPW = _NROWS // _NW
_PAIRS_PW = B * H // _NW


def _sc_gather(kf, vf, idxf):
    mesh = plsc.VectorSubcoreMesh(core_axis_name="c", subcore_axis_name="s")
    total = B * H * CACHE

    @functools.partial(
        pl.kernel, mesh=mesh,
        out_type=(jax.ShapeDtypeStruct((total, D), jnp.float32),
                  jax.ShapeDtypeStruct((total, D), jnp.float32)),
        scratch_types=[
            pltpu.VMEM((_RPW, LANES), jnp.int32),
            pltpu.VMEM((LANES, D), jnp.float32),
            pltpu.VMEM((LANES, D), jnp.float32),
            pltpu.VMEM((LANES, D), jnp.float32),
            pltpu.VMEM((LANES, D), jnp.float32),
            pltpu.SemaphoreType.DMA,
            pltpu.SemaphoreType.DMA,
            pltpu.SemaphoreType.DMA,
        ],
    )
    def body(k_hbm, v_hbm, idx_hbm, gk_hbm, gv_hbm,
             idx_v, bufka, bufva, bufkb, bufvb, sema, semb, semr):
        wid = lax.axis_index("s") * _NC + lax.axis_index("c")

        for j in range(_PAIRS_PW):
            p = wid * _PAIRS_PW + j
            pltpu.async_copy(k_hbm.at[pl.ds(p * S + SEL, RECENT)],
                             gk_hbm.at[pl.ds(p * CACHE + IMP_K, RECENT)],
                             semr)
            pltpu.async_copy(v_hbm.at[pl.ds(p * S + SEL, RECENT)],
                             gv_hbm.at[pl.ds(p * CACHE + IMP_K, RECENT)],
                             semr)

        base = wid * _RPW
        pltpu.sync_copy(idx_hbm.at[pl.ds(base, _RPW)], idx_v)

        def out_off(row):
            return (row >> 2) * CACHE + (row & 3) * LANES

        pltpu.async_copy(k_hbm.at[idx_v.at[0]], bufka, sema)
        pltpu.async_copy(v_hbm.at[idx_v.at[0]], bufva, sema)

        def pair_step(t, carry):
            ra = 2 * t
            rb = 2 * t + 1
            rn = jnp.minimum(2 * t + 2, _RPW - 1)
            ckb = pltpu.async_copy(k_hbm.at[idx_v.at[rb]], bufkb, semb)
            cvb = pltpu.async_copy(v_hbm.at[idx_v.at[rb]], bufvb, semb)
            cka = pltpu.make_async_copy(k_hbm.at[idx_v.at[ra]], bufka, sema)
            cva = pltpu.make_async_copy(v_hbm.at[idx_v.at[ra]], bufva, sema)
            cka.wait()
            cva.wait()
            oa = out_off(base + ra)
            pltpu.sync_copy(bufka, gk_hbm.at[pl.ds(oa, LANES)])
            pltpu.sync_copy(bufva, gv_hbm.at[pl.ds(oa, LANES)])
            pltpu.async_copy(k_hbm.at[idx_v.at[rn]], bufka, sema)
            pltpu.async_copy(v_hbm.at[idx_v.at[rn]], bufva, sema)
            ckb.wait()
            cvb.wait()
            ob = out_off(base + rb)
            pltpu.sync_copy(bufkb, gk_hbm.at[pl.ds(ob, LANES)])
            pltpu.sync_copy(bufvb, gv_hbm.at[pl.ds(ob, LANES)])
            return carry

        lax.fori_loop(0, _RPW // 2, pair_step, jnp.int32(0))
        pltpu.make_async_copy(k_hbm.at[idx_v.at[0]], bufka, sema).wait()
        pltpu.make_async_copy(v_hbm.at[idx_v.at[0]], bufva, sema).wait()
        for j in range(_PAIRS_PW):
            p = wid * _PAIRS_PW + j
            pltpu.make_async_copy(
                k_hbm.at[pl.ds(p * S + SEL, RECENT)],
                gk_hbm.at[pl.ds(p * CACHE + IMP_K, RECENT)], semr).wait()
            pltpu.make_async_copy(
                v_hbm.at[pl.ds(p * S + SEL, RECENT)],
                gv_hbm.at[pl.ds(p * CACHE + IMP_K, RECENT)], semr).wait()

    return body(kf, vf, idxf)


def kernel(k, v, attn_scores):
    attn_r = attn_scores.reshape(B, H, Q, ROWS, LANES)
    idx, imp, cnt = _tc_select(attn_r)
    gk, gv = _sc_gather(k.reshape(B * H * S, D),
                        v.reshape(B * H * S, D),
                        idx.reshape(_NROWS, LANES))
    return (gk.reshape(B, H, CACHE, D),
            gv.reshape(B, H, CACHE, D),
            imp.reshape(B, CACHE),
            cnt.reshape(B, CACHE))


# speedup vs baseline: 13.2732x; 13.2732x over previous
import functools

import jax
import jax.numpy as jnp
from jax import lax
from jax.experimental import pallas as pl
from jax.experimental.pallas import tpu as pltpu
from jax.experimental.pallas import tpu_sc as plsc

IMP_K = 512
RECENT = 512
CACHE = IMP_K + RECENT
B, H, Q, S, D = 8, 16, 4, 4096, 128
SEL = S - RECENT
ROWS, LANES = 32, 128
SROWS = SEL // LANES
KHI = IMP_K // LANES


def _tc_search_body(attn_ref, imp_ref, thr_ref):
    a = attn_ref[...]
    imp3d = jnp.sum(jnp.sum(a, axis=2), axis=1) / H
    imp_ref[...] = imp3d

    row_io = lax.broadcasted_iota(jnp.int32, (B, ROWS, LANES), 1)
    col_io = lax.broadcasted_iota(jnp.int32, (B, ROWS, LANES), 2)
    sel = row_io * LANES + col_io < SEL

    key = lax.bitcast_convert_type(imp3d, jnp.int32)
    keym = jnp.where(sel, key, jnp.int32(-1))

    def bit_step(i, t):
        cand = t | (jnp.int32(1) << (jnp.int32(30) - i))
        over = (keym >= cand[:, :, None]).astype(jnp.int32)
        cnt = jnp.sum(jnp.sum(over, axis=2), axis=1, keepdims=True)
        return jnp.where(cnt >= IMP_K, cand, t)

    thr = lax.fori_loop(0, 31, bit_step, jnp.zeros((B, 1), jnp.int32))
    thr_ref[...] = thr[:, :, None]


def _tc_select_body(imp_ref, thr_ref, idx_ref, impo_ref, cnt_ref):
    b = pl.program_id(0)
    imp2d = imp_ref[0]

    row_io = lax.broadcasted_iota(jnp.int32, (ROWS, LANES), 0)
    col_io = lax.broadcasted_iota(jnp.int32, (ROWS, LANES), 1)
    sidx = row_io * LANES + col_io
    sel = sidx < SEL

    key = lax.bitcast_convert_type(imp2d, jnp.int32)
    keym = jnp.where(sel, key, jnp.int32(-1))

    thr = thr_ref[0, 0, 0]

    gt = keym > thr
    eq = keym == thr
    n_eq_take = (jnp.int32(IMP_K) - jnp.sum(gt.astype(jnp.int32))).astype(
        jnp.float32)

    strict_u = (lax.broadcasted_iota(jnp.int32, (LANES, LANES), 0)
                < lax.broadcasted_iota(jnp.int32, (LANES, LANES), 1)
                ).astype(jnp.float32)
    row_lt = (lax.broadcasted_iota(jnp.int32, (ROWS, ROWS), 1)
              < lax.broadcasted_iota(jnp.int32, (ROWS, ROWS), 0)
              ).astype(jnp.float32)

    def eprefix(m):
        within = jnp.dot(m, strict_u, preferred_element_type=jnp.float32,
                         precision=lax.Precision.HIGHEST)
        offs = jnp.dot(row_lt, jnp.sum(m, axis=1, keepdims=True),
                       preferred_element_type=jnp.float32,
                       precision=lax.Precision.HIGHEST)
        return within + offs

    tie_rank = eprefix(eq.astype(jnp.float32))
    keep = gt | (eq & (tie_rank < n_eq_take))
    rank = eprefix(keep.astype(jnp.float32)).astype(jnp.int32)

    rank_hi = rank >> 7
    rank_lo = jnp.where(keep, rank & 127, jnp.int32(-1))
    rank_hi_f = rank_hi.astype(jnp.float32)
    rank_lo_f = rank_lo.astype(jnp.float32)

    key_b0 = (key & 255).astype(jnp.float32)
    key_b1 = ((key >> 8) & 255).astype(jnp.float32)
    key_b2 = ((key >> 16) & 255).astype(jnp.float32)
    key_b3 = ((key >> 24) & 127).astype(jnp.float32)
    col_f = col_io.astype(jnp.float32)

    jlo_io = (lax.broadcasted_iota(jnp.int32, (LANES, LANES), 0)
              ).astype(jnp.float32)

    accs = [jnp.zeros((6, LANES), jnp.float32) for _ in range(KHI)]
    for i in range(SROWS):
        rlo = rank_lo_f[i:i + 1, :]
        rhi = rank_hi_f[i:i + 1, :]
        oneh = jnp.where(jlo_io == rlo, 1.0, 0.0)
        pay = jnp.concatenate(
            [jnp.full((1, LANES), float(i), jnp.float32),
             col_f[i:i + 1, :],
             key_b0[i:i + 1, :], key_b1[i:i + 1, :],
             key_b2[i:i + 1, :], key_b3[i:i + 1, :]], axis=0)
        for g in range(KHI):
            pay_g = pay * jnp.where(rhi == float(g), 1.0, 0.0)
            accs[g] = accs[g] + lax.dot_general(
                pay_g, oneh, (((1,), (1,)), ((), ())),
                preferred_element_type=jnp.float32)

    ch = [jnp.concatenate([accs[g][c:c + 1, :] for g in range(KHI)], axis=0)
          for c in range(6)]
    kidx_top = (ch[0].astype(jnp.int32) * LANES + ch[1].astype(jnp.int32))
    keybits = (ch[2].astype(jnp.int32)
               | (ch[3].astype(jnp.int32) << 8)
               | (ch[4].astype(jnp.int32) << 16)
               | (ch[5].astype(jnp.int32) << 24))
    imp_top = lax.bitcast_convert_type(keybits, jnp.float32)

    kidx8 = jnp.concatenate([kidx_top, sidx[SROWS:, :]], axis=0)
    imp8 = jnp.concatenate([imp_top, imp2d[SROWS:, :]], axis=0)

    h_io = lax.broadcasted_iota(jnp.int32, (H, 1, 1), 0)
    flat = (b * H + h_io) * S + kidx8[None]
    idx_ref[...] = flat[None]
    impo_ref[...] = imp8[None]
    cnt_ref[...] = (jnp.float32(S) - kidx8.astype(jnp.float32))[None]


def _tc_select(attn_r):
    imp_all, thr_all = pl.pallas_call(
        _tc_search_body,
        out_shape=[
            jax.ShapeDtypeStruct((B, ROWS, LANES), jnp.float32),
            jax.ShapeDtypeStruct((B, 1, 1), jnp.int32),
        ],
    )(attn_r)
    return pl.pallas_call(
        _tc_select_body,
        grid=(B,),
        in_specs=[
            pl.BlockSpec((1, ROWS, LANES), lambda b: (b, 0, 0)),
            pl.BlockSpec((1, 1, 1), lambda b: (b, 0, 0),
                         memory_space=pltpu.SMEM),
        ],
        out_specs=[
            pl.BlockSpec((1, H, CACHE // LANES, LANES),
                         lambda b: (b, 0, 0, 0)),
            pl.BlockSpec((1, CACHE // LANES, LANES), lambda b: (b, 0, 0)),
            pl.BlockSpec((1, CACHE // LANES, LANES), lambda b: (b, 0, 0)),
        ],
        out_shape=[
            jax.ShapeDtypeStruct((B, H, CACHE // LANES, LANES), jnp.int32),
            jax.ShapeDtypeStruct((B, CACHE // LANES, LANES), jnp.float32),
            jax.ShapeDtypeStruct((B, CACHE // LANES, LANES), jnp.float32),
        ],
    )(imp_all, thr_all)


_NC, _NS = 2, 16
_NW = _NC * _NS
_NROWS = B * H * CACHE // LANES
_RPW = _NROWS // _NW


def _sc_gather(kf, vf, idxf):
    mesh = plsc.VectorSubcoreMesh(core_axis_name="c", subcore_axis_name="s")
    total = B * H * CACHE

    @functools.partial(
        pl.kernel, mesh=mesh,
        out_type=(jax.ShapeDtypeStruct((total, D), jnp.float32),
                  jax.ShapeDtypeStruct((total, D), jnp.float32)),
        scratch_types=[
            pltpu.VMEM((_RPW, LANES), jnp.int32),
            pltpu.VMEM((LANES, D), jnp.float32),
            pltpu.VMEM((LANES, D), jnp.float32),
            pltpu.VMEM((LANES, D), jnp.float32),
            pltpu.VMEM((LANES, D), jnp.float32),
            pltpu.SemaphoreType.DMA,
            pltpu.SemaphoreType.DMA,
        ],
    )
    def body(k_hbm, v_hbm, idx_hbm, gk_hbm, gv_hbm,
             idx_v, bufka, bufva, bufkb, bufvb, sema, semb):
        wid = lax.axis_index("s") * _NC + lax.axis_index("c")
        base = wid * _RPW
        pltpu.sync_copy(idx_hbm.at[pl.ds(base, _RPW)], idx_v)

        k0 = pltpu.async_copy(k_hbm.at[idx_v.at[0]], bufka, sema)
        v0 = pltpu.async_copy(v_hbm.at[idx_v.at[0]], bufva, sema)

        def pair(t, carry):
            ra = 2 * t
            rb = 2 * t + 1
            rn = jnp.minimum(2 * t + 2, _RPW - 1)
            ckb = pltpu.async_copy(k_hbm.at[idx_v.at[rb]], bufkb, semb)
            cvb = pltpu.async_copy(v_hbm.at[idx_v.at[rb]], bufvb, semb)
            cka = pltpu.make_async_copy(k_hbm.at[idx_v.at[ra]], bufka, sema)
            cva = pltpu.make_async_copy(v_hbm.at[idx_v.at[ra]], bufva, sema)
            cka.wait()
            cva.wait()
            pltpu.sync_copy(bufka, gk_hbm.at[pl.ds((base + ra) * LANES,
                                                   LANES)])
            pltpu.sync_copy(bufva, gv_hbm.at[pl.ds((base + ra) * LANES,
                                                   LANES)])
            pltpu.async_copy(k_hbm.at[idx_v.at[rn]], bufka, sema)
            pltpu.async_copy(v_hbm.at[idx_v.at[rn]], bufva, sema)
            ckb.wait()
            cvb.wait()
            pltpu.sync_copy(bufkb, gk_hbm.at[pl.ds((base + rb) * LANES,
                                                   LANES)])
            pltpu.sync_copy(bufvb, gv_hbm.at[pl.ds((base + rb) * LANES,
                                                   LANES)])
            return carry

        lax.fori_loop(0, _RPW // 2, pair, jnp.int32(0))
        pltpu.make_async_copy(k_hbm.at[idx_v.at[0]], bufka, sema).wait()
        pltpu.make_async_copy(v_hbm.at[idx_v.at[0]], bufva, sema).wait()

    return body(kf, vf, idxf)


def kernel(k, v, attn_scores):
    attn_r = attn_scores.reshape(B, H, Q, ROWS, LANES)
    idx, imp, cnt = _tc_select(attn_r)
    gk, gv = _sc_gather(k.reshape(B * H * S, D),
                        v.reshape(B * H * S, D),
                        idx.reshape(_NROWS, LANES))
    return (gk.reshape(B, H, CACHE, D),
            gv.reshape(B, H, CACHE, D),
            imp.reshape(B, CACHE),
            cnt.reshape(B, CACHE))
